# baseline (speedup 1.0000x reference)
import functools

import jax
import jax.numpy as jnp
from jax import lax
from jax.experimental import pallas as pl
from jax.experimental.pallas import tpu as pltpu
from jax.experimental.pallas import tpu_sc as plsc

B, T, H, K = 4, 4096, 1024, 512
TC = 512
NW = 32
TPW = T * B // NW
CPB = T // TPW
GRP = 16
DUMMY_ROW = B * K
OUT_ROWS = B * K + 8


def _scores_body(h_ref, w_ref, o_ref):
    h = h_ref[0].astype(jnp.bfloat16)
    w = w_ref[...].astype(jnp.bfloat16)
    o_ref[0, 0, :] = jnp.dot(h, w.T, preferred_element_type=jnp.float32)[:, 0]


def _scores_call(hidden, W):
    return pl.pallas_call(
        _scores_body,
        grid=(B, T // TC),
        in_specs=[
            pl.BlockSpec((1, TC, H), lambda b, c: (b, c, 0)),
            pl.BlockSpec((1, H), lambda b, c: (0, 0)),
        ],
        out_specs=pl.BlockSpec((1, 1, TC), lambda b, c: (b, 0, c)),
        out_shape=jax.ShapeDtypeStruct((B, 1, T), jnp.float32),
    )(hidden, W)


def _prefix_sum(x):
    n = x.shape[1]
    acc = x
    sh = 1
    while sh < n:
        pad = jnp.zeros((x.shape[0], sh), acc.dtype)
        acc = acc + jnp.concatenate([pad, acc[:, : n - sh]], axis=1)
        sh *= 2
    return acc - x


def _select_body(s_ref, enc_ref, meta_ref):
    s = s_ref[...]
    bits = lax.bitcast_convert_type(s, jnp.int32)
    key = jnp.where(bits < 0, bits ^ jnp.int32(0x7FFFFFFF), bits)
    int_min = jnp.int32(-(2 ** 31))

    def search(i, tpat):
        bit = jnp.int32(31) - i
        cand = tpat | jnp.left_shift(jnp.int32(1), bit)
        thr = cand ^ int_min
        cnt = jnp.sum((key >= thr).astype(jnp.int32), axis=1, keepdims=True)
        return jnp.where(cnt >= K, cand, tpat)

    tpat = lax.fori_loop(0, 32, search, jnp.zeros((B, 1), jnp.int32))
    thr = tpat ^ int_min
    gt = key > thr
    eq = key == thr
    n_gt = jnp.sum(gt.astype(jnp.int32), axis=1, keepdims=True)
    need = K - n_gt
    eq_rank = _prefix_sum(eq.astype(jnp.int32))
    sel = gt | (eq & (eq_rank < need))

    t_loc = lax.broadcasted_iota(jnp.int32, (B, T), 1)
    b_ids = lax.broadcasted_iota(jnp.int32, (B, T), 0)
    enc_ref[...] = jnp.where(sel, t_loc + b_ids * T, jnp.int32(-1))

    sel_f = sel.astype(jnp.float32)
    tc_id = lax.broadcasted_iota(jnp.int32, (T, CPB), 0) // TPW
    c_id = lax.broadcasted_iota(jnp.int32, (T, CPB), 1)
    e_mat = (tc_id == c_id).astype(jnp.float32)
    cnts = jax.lax.dot(sel_f, e_mat,
                       precision=lax.Precision.HIGHEST)
    lo = lax.broadcasted_iota(jnp.int32, (CPB, CPB), 0)
    hi = lax.broadcasted_iota(jnp.int32, (CPB, CPB), 1)
    tri = (lo < hi).astype(jnp.float32)
    pstart = jax.lax.dot(cnts, tri,
                         precision=lax.Precision.HIGHEST)
    b_off = lax.broadcasted_iota(jnp.int32, (B, CPB), 0) * K
    meta_ref[...] = pstart.astype(jnp.int32) + b_off


def _select_call(scores):
    return pl.pallas_call(
        _select_body,
        out_shape=[
            jax.ShapeDtypeStruct((B, T), jnp.int32),
            jax.ShapeDtypeStruct((B, CPB), jnp.int32),
        ],
    )(scores)


def _gather_body(enc_hbm, meta_hbm, table_hbm, out_hbm,
                 enc_v, meta_v, cidx, tbuf, obuf, rows, sem_g, sem_s):
    nc = 2
    w = lax.axis_index("s") * nc + lax.axis_index("c")
    pltpu.sync_copy(enc_hbm.at[pl.ds(w * TPW, TPW)], enc_v)
    pltpu.sync_copy(meta_hbm, meta_v)

    lanes = lax.iota(jnp.int32, 16)
    zero16 = jnp.zeros((16,), jnp.int32)
    v0 = meta_v[pl.ds(0, 16)]
    v1 = meta_v[pl.ds(16, 16)]
    c0 = jnp.where(lanes == w, v0, zero16)
    c1 = jnp.where(lanes + 16 == w, v1, zero16)
    pstart = jnp.sum(c0) + jnp.sum(c1)

    zero = jnp.zeros((16,), jnp.int32)
    for i in range((TPW + GRP + 15) // 16):
        cidx[pl.ds(i * 16, 16)] = zero

    def compact(i, cnt):
        v = enc_v[pl.ds(i * 16, 16)]
        m = v >= 0
        plsc.store_compressed(cidx.at[pl.ds(cnt, 16)], v, mask=m)
        return cnt + jnp.sum(m.astype(jnp.int32))

    cnt = lax.fori_loop(0, TPW // 16, compact, jnp.int32(0))

    for g in range(TPW // GRP):
        @pl.when(cnt > g * GRP)
        def _():
            tbuf[...] = cidx[pl.ds(g * GRP, GRP)]
            opos = pstart + g * GRP + lanes
            valid = (g * GRP + lanes) < cnt
            obuf[...] = jnp.where(valid, opos, jnp.int32(DUMMY_ROW))
            pltpu.async_copy(table_hbm.at[tbuf], rows, sem_g).wait()
            pltpu.async_copy(rows, out_hbm.at[obuf], sem_s).wait()


def _gather_call(enc_flat, meta_flat, table):
    mesh = plsc.VectorSubcoreMesh(core_axis_name="c", subcore_axis_name="s")
    fn = functools.partial(
        pl.kernel,
        out_type=jax.ShapeDtypeStruct((OUT_ROWS, H), jnp.float32),
        mesh=mesh,
        compiler_params=pltpu.CompilerParams(needs_layout_passes=False),
        scratch_types=[
            pltpu.VMEM((TPW,), jnp.int32),
            pltpu.VMEM((NW,), jnp.int32),
            pltpu.VMEM((TPW + GRP,), jnp.int32),
            pltpu.VMEM((GRP,), jnp.int32),
            pltpu.VMEM((GRP,), jnp.int32),
            pltpu.VMEM((GRP, H), jnp.float32),
            pltpu.SemaphoreType.DMA,
            pltpu.SemaphoreType.DMA,
        ],
    )(_gather_body)
    return fn(enc_flat, meta_flat, table)


def kernel(hidden, W, b):
    del b
    scores = _scores_call(hidden, W).reshape(B, T)
    enc, meta = _select_call(scores)
    out = _gather_call(enc.reshape(B * T), meta.reshape(NW),
                       hidden.reshape(B * T, H))
    memory = out[: B * K].reshape(B, K, H)
    mask = jnp.ones((B, K), hidden.dtype)
    return memory, mask
